# Initial kernel scaffold; baseline (speedup 1.0000x reference)
import functools

import jax
import jax.numpy as jnp
from jax import lax
from jax.experimental import pallas as pl
from jax.experimental.pallas import tpu as pltpu
from jax.experimental.pallas import tpu_sc as plsc

NUM_CORES = 2
NUM_SUBCORES = 16
LANES = 16


@functools.partial(jax.jit, static_argnames=("seq_len", "seqs_per_chunk"))
def _sc_embed(xf, table, pos_emb, *, seq_len, seqs_per_chunk):
    n = xf.shape[0]
    d = table.shape[1]
    nw = NUM_CORES * NUM_SUBCORES
    per_w = n // nw
    chunk = seqs_per_chunk * seq_len
    n_chunks = per_w // chunk
    assert per_w % chunk == 0 and n % nw == 0 and d % LANES == 0
    vregs_per_row = d // LANES

    mesh = plsc.VectorSubcoreMesh(
        core_axis_name="c", subcore_axis_name="s",
        num_cores=NUM_CORES, num_subcores=NUM_SUBCORES,
    )

    @functools.partial(
        pl.kernel,
        mesh=mesh,
        out_type=jax.ShapeDtypeStruct((n, d), jnp.float32),
        scratch_types=[
            pltpu.VMEM((chunk,), jnp.int32),
            pltpu.VMEM((chunk, d), jnp.float32),
            pltpu.VMEM((seq_len, d), jnp.float32),
            pltpu.SemaphoreType.DMA,
        ],
    )
    def k(x_hbm, table_hbm, pos_hbm, out_hbm, idx_v, rows_v, pos_v, sem):
        wid = lax.axis_index("s") * NUM_CORES + lax.axis_index("c")
        base = wid * per_w
        pltpu.sync_copy(pos_hbm, pos_v)

        def do_chunk(c, carry):
            off = base + c * chunk
            pltpu.sync_copy(x_hbm.at[pl.ds(off, chunk)], idx_v)
            pltpu.async_copy(table_hbm.at[idx_v], rows_v, sem).wait()

            def add_pos(l, carry2):
                for v in range(vregs_per_row):
                    p = pos_v[l, pl.ds(v * LANES, LANES)]
                    for s in range(seqs_per_chunk):
                        r = s * seq_len + l
                        rows_v[r, pl.ds(v * LANES, LANES)] += p
                return carry2

            lax.fori_loop(0, seq_len, add_pos, 0, unroll=False)
            pltpu.sync_copy(rows_v, out_hbm.at[pl.ds(off, chunk)])
            return carry

        lax.fori_loop(0, n_chunks, do_chunk, 0, unroll=False)

    return k(xf, table, pos_emb)


def kernel(x, table, pos_emb):
    b, l = x.shape
    xf = x.reshape(-1).astype(jnp.int32)
    out = _sc_embed(xf, table, pos_emb[:l], seq_len=l, seqs_per_chunk=4)
    return out.reshape(b, l, table.shape[1])


# speedup vs baseline: 1.3929x; 1.3929x over previous
import functools

import jax
import jax.numpy as jnp
from jax import lax
from jax.experimental import pallas as pl
from jax.experimental.pallas import tpu as pltpu
from jax.experimental.pallas import tpu_sc as plsc

NUM_CORES = 2
NUM_SUBCORES = 16
LANES = 16


@functools.partial(jax.jit, static_argnames=("seq_len", "seqs_per_chunk"))
def _sc_embed(xf, table, pos_emb, *, seq_len, seqs_per_chunk):
    n = xf.shape[0]
    d = table.shape[1]
    nw = NUM_CORES * NUM_SUBCORES
    per_w = n // nw
    chunk = seqs_per_chunk * seq_len
    n_chunks = per_w // chunk
    assert per_w % chunk == 0 and n % nw == 0 and d % LANES == 0
    vregs_per_row = d // LANES

    mesh = plsc.VectorSubcoreMesh(
        core_axis_name="c", subcore_axis_name="s",
        num_cores=NUM_CORES, num_subcores=NUM_SUBCORES,
    )

    @functools.partial(
        pl.kernel,
        mesh=mesh,
        out_type=jax.ShapeDtypeStruct((n, d), jnp.float32),
        scratch_types=[
            pltpu.VMEM((chunk,), jnp.int32),
            pltpu.VMEM((chunk, d), jnp.float32),
            pltpu.VMEM((seq_len, d), jnp.float32),
            pltpu.SemaphoreType.DMA,
        ],
        compiler_params=pltpu.CompilerParams(use_tc_tiling_on_sc=False),
    )
    def k(x_hbm, table_hbm, pos_hbm, out_hbm, idx_v, rows_v, pos_v, sem):
        wid = lax.axis_index("s") * NUM_CORES + lax.axis_index("c")
        base = wid * per_w
        pltpu.sync_copy(pos_hbm, pos_v)

        def do_chunk(c, carry):
            off = base + c * chunk
            pltpu.sync_copy(x_hbm.at[pl.ds(off, chunk)], idx_v)
            pltpu.async_copy(table_hbm.at[idx_v], rows_v, sem).wait()

            def add_pos(l, carry2):
                for v in range(vregs_per_row):
                    p = pos_v[l, pl.ds(v * LANES, LANES)]
                    for s in range(seqs_per_chunk):
                        r = s * seq_len + l
                        rows_v[r, pl.ds(v * LANES, LANES)] += p
                return carry2

            lax.fori_loop(0, seq_len, add_pos, 0, unroll=False)
            pltpu.sync_copy(rows_v, out_hbm.at[pl.ds(off, chunk)])
            return carry

        lax.fori_loop(0, n_chunks, do_chunk, 0, unroll=False)

    return k(xf, table, pos_emb)


def kernel(x, table, pos_emb):
    b, l = x.shape
    xf = x.reshape(-1).astype(jnp.int32)
    out = _sc_embed(xf, table, pos_emb[:l], seq_len=l, seqs_per_chunk=4)
    return out.reshape(b, l, table.shape[1])
